# baseline (speedup 1.0000x reference)
import functools
import math

import jax
import jax.numpy as jnp
import numpy as np
from jax import lax
from jax.experimental import pallas as pl
from jax.experimental.pallas import tpu as pltpu
from jax.experimental.pallas import tpu_sc as plsc

N_ATOMS = 10000
N_EDGES = 320000
D = 128
NF = 128
NRBF = 20
MAXZ = 101
CUTOFF = 5.0

SC_CORES = 2
SC_SUBCORES = 16
EDGES_PER_CORE = N_EDGES // SC_CORES
EDGES_PER_SUB = EDGES_PER_CORE // SC_SUBCORES
BE = 80
NBLK = EDGES_PER_SUB // BE
ROWS_PER_SUB = 624
ROWS_TAIL = N_ATOMS - ROWS_PER_SUB * SC_SUBCORES

BN = 1000
BE_TC = 4000

_HI = jax.lax.Precision.HIGHEST
_DEF = jax.lax.Precision.DEFAULT
_MEGACORE = pltpu.CompilerParams(dimension_semantics=("parallel",))


def _ssp(x):
    return jnp.maximum(x, 0.0) + jnp.log(1.0 + jnp.exp(-jnp.abs(x))) - np.float32(np.log(2.0))


def _embed_body(z_ref, emb_ref, w_ref, x_ref, h_ref):
    z = z_ref[0, 0, :]
    oh = (z[:, None] == lax.broadcasted_iota(jnp.int32, (BN, MAXZ), 1)).astype(jnp.float32)
    x = jnp.dot(oh, emb_ref[...], precision=_HI)
    x_ref[...] = x
    h_ref[...] = jnp.dot(x, w_ref[...], precision=_HI)


def _embed_call(z3, emb, w0):
    grid = (N_ATOMS // BN,)
    return pl.pallas_call(
        _embed_body,
        grid=grid,
        in_specs=[
            pl.BlockSpec((1, 1, BN), lambda i: (i, 0, 0)),
            pl.BlockSpec((MAXZ, D), lambda i: (0, 0)),
            pl.BlockSpec((D, NF), lambda i: (0, 0)),
        ],
        out_specs=[
            pl.BlockSpec((BN, D), lambda i: (i, 0)),
            pl.BlockSpec((BN, NF), lambda i: (i, 0)),
        ],
        out_shape=[
            jax.ShapeDtypeStruct((N_ATOMS, D), jnp.float32),
            jax.ShapeDtypeStruct((N_ATOMS, NF), jnp.float32),
        ],
        compiler_params=_MEGACORE,
    )(z3, emb, w0)


_RCUT_COEF = tuple(np.float32(v) for v in (
    1.0, -4.93480110168457, 4.058694839477539, -1.3351584672927856,
    0.23502980172634125, -0.025358984246850014, 0.0015939107397571206,
))


def _wij_body(r_ref, w1_ref, b1_ref, w2_ref, b2_ref, o_ref):
    r = r_ref[...]
    d2 = jnp.sum(r * r, axis=1, keepdims=True)
    d = jnp.sqrt(d2)
    width = np.float32(np.float32(CUTOFF) / (NRBF - 1))
    coeff = np.float32(-0.5 / (width * width))
    offsets = lax.broadcasted_iota(jnp.int32, (1, NRBF), 1).astype(jnp.float32) * width
    f = jnp.exp(coeff * (d - offsets) ** 2)
    t = _ssp(jnp.dot(f, w1_ref[...], precision=_DEF) + b1_ref[...])
    t = jnp.dot(t, w2_ref[...], precision=_DEF) + b2_ref[...]
    z = d2 * np.float32(1.0 / (CUTOFF * CUTOFF))
    p = jnp.full_like(z, _RCUT_COEF[-1])
    for c in _RCUT_COEF[-2::-1]:
        p = p * z + c
    rcut = 0.5 * (p + 1.0) * (z < 1.0).astype(jnp.float32)
    v = t * rcut
    lo = lax.bitcast_convert_type(v[:, : NF // 2].astype(jnp.bfloat16), jnp.uint16)
    hi = lax.bitcast_convert_type(v[:, NF // 2:].astype(jnp.bfloat16), jnp.uint16)
    o_ref[...] = ((hi.astype(jnp.uint32) << 16) | lo.astype(jnp.uint32)).astype(jnp.int32)


def _wij_call(r_ij, w1, b1, w2, b2):
    grid = (N_EDGES // BE_TC,)
    return pl.pallas_call(
        _wij_body,
        grid=grid,
        in_specs=[
            pl.BlockSpec((BE_TC, 3), lambda i: (i, 0)),
            pl.BlockSpec((NRBF, NF), lambda i: (0, 0)),
            pl.BlockSpec((1, NF), lambda i: (0, 0)),
            pl.BlockSpec((NF, NF), lambda i: (0, 0)),
            pl.BlockSpec((1, NF), lambda i: (0, 0)),
        ],
        out_specs=pl.BlockSpec((BE_TC, NF // 2), lambda i: (i, 0)),
        out_shape=jax.ShapeDtypeStruct((N_EDGES, NF // 2), jnp.int32),
        compiler_params=_MEGACORE,
    )(r_ij, w1, b1.reshape(1, NF), w2, b2.reshape(1, NF))


_WIJ_PERM = np.empty((NF,), np.int32)
for _g in range(NF // 32):
    for _t in range(16):
        _WIJ_PERM[_g * 16 + _t] = _g * 32 + _t
        _WIJ_PERM[NF // 2 + _g * 16 + _t] = _g * 32 + 16 + _t


@functools.lru_cache(maxsize=None)
def _make_sc_fused():
    mesh = plsc.VectorSubcoreMesh(
        core_axis_name="c", subcore_axis_name="s",
        num_cores=SC_CORES, num_subcores=SC_SUBCORES,
    )
    cp = pltpu.CompilerParams()
    if "needs_layout_passes" in pltpu.CompilerParams.__dataclass_fields__:
        import dataclasses as _dc
        cp = _dc.replace(cp, needs_layout_passes=False)
    return pl.kernel(
        _sc_fused_body,
        out_type=jax.ShapeDtypeStruct((SC_CORES, N_ATOMS, D), jnp.float32),
        mesh=mesh,
        compiler_params=cp,
        scratch_types=[
            pltpu.VMEM_SHARED((N_ATOMS, D), jnp.float32),
            pltpu.VMEM((BE,), jnp.int32),
            pltpu.VMEM((BE,), jnp.int32),
            pltpu.VMEM((BE,), jnp.int32),
            pltpu.VMEM((BE,), jnp.int32),
            pltpu.VMEM((BE, D), jnp.float32),
            pltpu.VMEM((BE, D), jnp.float32),
            pltpu.VMEM((BE, D // 2), jnp.int32),
            pltpu.VMEM((BE, D // 2), jnp.int32),
            pltpu.SemaphoreType.DMA,
            pltpu.SemaphoreType.DMA,
            pltpu.SemaphoreType.DMA,
            pltpu.SemaphoreType.DMA,
            pltpu.SemaphoreType.DMA,
            pltpu.SemaphoreType.DMA,
        ],
    )


def _sc_call(h, wij, idx_i, idx_j, zeros):
    wij4 = wij.reshape(SC_CORES, SC_SUBCORES, NBLK, BE, NF // 2)
    idxi4 = idx_i.reshape(SC_CORES, SC_SUBCORES, NBLK, BE)
    idxj4 = idx_j.reshape(SC_CORES, SC_SUBCORES, NBLK, BE)
    return _make_sc_fused()(h, wij4, idxi4, idxj4, zeros)


def _sc_fused_body(h_hbm, wij_hbm, idxi_hbm, idxj_hbm, zero_hbm, out_hbm,
                   agg_sh, idxi0, idxi1, idxj0, idxj1,
                   rows0, rows1, wijb0, wijb1,
                   sg0, sg1, sw0, sw1, si0, si1):
    cid = lax.axis_index("c")
    sid = lax.axis_index("s")
    idxi = (idxi0, idxi1)
    idxj = (idxj0, idxj1)
    rows = (rows0, rows1)
    wijb = (wijb0, wijb1)
    sg = (sg0, sg1)
    sw = (sw0, sw1)
    si = (si0, si1)

    pltpu.sync_copy(zero_hbm, agg_sh.at[pl.ds(sid * ROWS_PER_SUB, ROWS_PER_SUB)])

    @pl.when(sid == SC_SUBCORES - 1)
    def _zero_tail():
        pltpu.sync_copy(zero_hbm.at[pl.ds(0, ROWS_TAIL)],
                        agg_sh.at[pl.ds(SC_SUBCORES * ROWS_PER_SUB, ROWS_TAIL)])

    plsc.subcore_barrier()

    def start_idx(blk, p):
        pltpu.async_copy(idxi_hbm.at[cid, sid, blk], idxi[p], si[p])
        pltpu.async_copy(idxj_hbm.at[cid, sid, blk], idxj[p], si[p])

    def wait_idx(blk, p):
        pltpu.make_async_copy(idxi_hbm.at[cid, sid, blk], idxi[p], si[p]).wait()
        pltpu.make_async_copy(idxj_hbm.at[cid, sid, blk], idxj[p], si[p]).wait()

    def start_gw(blk, p):
        pltpu.async_copy(h_hbm.at[idxj[p]], rows[p], sg[p])
        pltpu.async_copy(wij_hbm.at[cid, sid, blk], wijb[p], sw[p])

    def wait_gw(blk, p):
        pltpu.make_async_copy(h_hbm.at[idxj[p]], rows[p], sg[p]).wait()
        pltpu.make_async_copy(wij_hbm.at[cid, sid, blk], wijb[p], sw[p]).wait()

    def process(blk, p, static_last=False):
        if not static_last:
            @pl.when(blk + 1 < NBLK)
            def _gw_next():
                wait_idx(blk + 1, 1 - p)
                start_gw(blk + 1, 1 - p)

        wait_gw(blk, p)

        @plsc.parallel_loop(0, BE, unroll=2)
        def _row(i):
            for j in range(D // 32):
                wi = wijb[p][i, pl.ds(j * 16, 16)]
                w = plsc.bitcast(wi, jnp.bfloat16)
                wa, wb = plsc.unpack(w, format=plsc.PackFormat.INTERLEAVED)
                sa = (i, pl.ds(j * 32, 16))
                sb = (i, pl.ds(j * 32 + 16, 16))
                rows[p][sa] = rows[p][sa] * wa
                rows[p][sb] = rows[p][sb] * wb

        pltpu.sync_copy(rows[p], agg_sh.at[idxi[p]], add=True)

        if not static_last:
            @pl.when(blk + 2 < NBLK)
            def _idx_next():
                start_idx(blk + 2, p)

    start_idx(0, 0)
    start_idx(1, 1)
    wait_idx(0, 0)
    start_gw(0, 0)

    @pl.loop(0, NBLK - 1, step=2)
    def _block(b):
        for p in range(2):
            process(b + p, p)

    process(NBLK - 1, 0, static_last=True)
    plsc.subcore_barrier()
    pltpu.sync_copy(
        agg_sh.at[pl.ds(sid * ROWS_PER_SUB, ROWS_PER_SUB)],
        out_hbm.at[cid, pl.ds(sid * ROWS_PER_SUB, ROWS_PER_SUB)],
    )

    @pl.when(sid == SC_SUBCORES - 1)
    def _write_tail():
        pltpu.sync_copy(
            agg_sh.at[pl.ds(SC_SUBCORES * ROWS_PER_SUB, ROWS_TAIL)],
            out_hbm.at[cid, pl.ds(SC_SUBCORES * ROWS_PER_SUB, ROWS_TAIL)],
        )


def _out_body(p_ref, x_ref, w1_ref, b1_ref, w2_ref, b2_ref, wn_ref, xo_ref, ho_ref):
    agg = p_ref[0] + p_ref[1]
    t = _ssp(jnp.dot(agg, w1_ref[...], precision=_HI) + b1_ref[...])
    v = jnp.dot(t, w2_ref[...], precision=_HI) + b2_ref[...]
    xn = x_ref[...] + v
    xo_ref[...] = xn
    ho_ref[...] = jnp.dot(xn, wn_ref[...], precision=_HI)


def _out_last_body(p_ref, x_ref, w1_ref, b1_ref, w2_ref, b2_ref, xo_ref):
    agg = p_ref[0] + p_ref[1]
    t = _ssp(jnp.dot(agg, w1_ref[...], precision=_HI) + b1_ref[...])
    v = jnp.dot(t, w2_ref[...], precision=_HI) + b2_ref[...]
    xo_ref[...] = x_ref[...] + v


def _out_call(partials, x, w1, b1, w2, b2, wn):
    grid = (N_ATOMS // BN,)
    common_in = [
        pl.BlockSpec((SC_CORES, BN, D), lambda i: (0, i, 0)),
        pl.BlockSpec((BN, D), lambda i: (i, 0)),
        pl.BlockSpec((NF, D), lambda i: (0, 0)),
        pl.BlockSpec((1, D), lambda i: (0, 0)),
        pl.BlockSpec((D, D), lambda i: (0, 0)),
        pl.BlockSpec((1, D), lambda i: (0, 0)),
    ]
    if wn is None:
        return pl.pallas_call(
            _out_last_body,
            grid=grid,
            in_specs=common_in,
            out_specs=pl.BlockSpec((BN, D), lambda i: (i, 0)),
            out_shape=jax.ShapeDtypeStruct((N_ATOMS, D), jnp.float32),
            compiler_params=_MEGACORE,
        )(partials, x, w1, b1.reshape(1, D), w2, b2.reshape(1, D))
    return pl.pallas_call(
        _out_body,
        grid=grid,
        in_specs=common_in + [pl.BlockSpec((D, NF), lambda i: (0, 0))],
        out_specs=[
            pl.BlockSpec((BN, D), lambda i: (i, 0)),
            pl.BlockSpec((BN, NF), lambda i: (i, 0)),
        ],
        out_shape=[
            jax.ShapeDtypeStruct((N_ATOMS, D), jnp.float32),
            jax.ShapeDtypeStruct((N_ATOMS, NF), jnp.float32),
        ],
        compiler_params=_MEGACORE,
    )(partials, x, w1, b1.reshape(1, D), w2, b2.reshape(1, D), wn)


def kernel(atomic_numbers, r_ij, idx_i, idx_j, emb,
           W_in2f, W_fn1, b_fn1, W_fn2, b_fn2,
           W_f2o1, b_f2o1, W_f2o2, b_f2o2):
    L = W_in2f.shape[0]
    z3 = atomic_numbers.astype(jnp.int32).reshape(N_ATOMS // BN, 1, BN)
    idx_i = idx_i.astype(jnp.int32)
    idx_j = idx_j.astype(jnp.int32)
    zeros = jnp.zeros((ROWS_PER_SUB, D), jnp.float32)

    x, h = _embed_call(z3, emb, W_in2f[0])
    for l in range(L):
        wij = _wij_call(r_ij, W_fn1[l], b_fn1[l],
                        W_fn2[l][:, _WIJ_PERM], b_fn2[l][_WIJ_PERM])
        partials = _sc_call(h, wij, idx_i, idx_j, zeros)
        if l < L - 1:
            x, h = _out_call(partials, x, W_f2o1[l], b_f2o1[l],
                             W_f2o2[l], b_f2o2[l], W_in2f[l + 1])
        else:
            x = _out_call(partials, x, W_f2o1[l], b_f2o1[l],
                          W_f2o2[l], b_f2o2[l], None)
    return x


# speedup vs baseline: 1.1444x; 1.1444x over previous
import functools
import math

import jax
import jax.numpy as jnp
import numpy as np
from jax import lax
from jax.experimental import pallas as pl
from jax.experimental.pallas import tpu as pltpu
from jax.experimental.pallas import tpu_sc as plsc

N_ATOMS = 10000
N_EDGES = 320000
D = 128
NF = 128
NRBF = 20
MAXZ = 101
CUTOFF = 5.0

SC_CORES = 2
SC_SUBCORES = 16
EDGES_PER_CORE = N_EDGES // SC_CORES
EDGES_PER_SUB = EDGES_PER_CORE // SC_SUBCORES
BE = 80
NBLK = EDGES_PER_SUB // BE
ROWS_PER_SUB = 624
ROWS_TAIL = N_ATOMS - ROWS_PER_SUB * SC_SUBCORES

BN = 1000
BE_TC = 4000

_HI = jax.lax.Precision.HIGHEST
_DEF = jax.lax.Precision.DEFAULT
_MEGACORE = pltpu.CompilerParams(dimension_semantics=("parallel",))


def _ssp(x):
    return jnp.maximum(x, 0.0) + jnp.log(1.0 + jnp.exp(-jnp.abs(x))) - np.float32(np.log(2.0))


def _embed_body(z_ref, emb_ref, w_ref, x_ref, h_ref):
    z = z_ref[0, 0, :]
    oh = (z[:, None] == lax.broadcasted_iota(jnp.int32, (BN, MAXZ), 1)).astype(jnp.float32)
    x = jnp.dot(oh, emb_ref[...], precision=_HI)
    x_ref[...] = x
    h_ref[...] = jnp.dot(x, w_ref[...], precision=_HI)


def _embed_call(z3, emb, w0):
    grid = (N_ATOMS // BN,)
    return pl.pallas_call(
        _embed_body,
        grid=grid,
        in_specs=[
            pl.BlockSpec((1, 1, BN), lambda i: (i, 0, 0)),
            pl.BlockSpec((MAXZ, D), lambda i: (0, 0)),
            pl.BlockSpec((D, NF), lambda i: (0, 0)),
        ],
        out_specs=[
            pl.BlockSpec((BN, D), lambda i: (i, 0)),
            pl.BlockSpec((BN, NF), lambda i: (i, 0)),
        ],
        out_shape=[
            jax.ShapeDtypeStruct((N_ATOMS, D), jnp.float32),
            jax.ShapeDtypeStruct((N_ATOMS, NF), jnp.float32),
        ],
        compiler_params=_MEGACORE,
    )(z3, emb, w0)


_RCUT_COEF = tuple(np.float32(v) for v in (
    1.0, -4.93480110168457, 4.058694839477539, -1.3351584672927856,
    0.23502980172634125, -0.025358984246850014, 0.0015939107397571206,
))


def _wij_body(r_ref, w1_ref, b1_ref, w2_ref, b2_ref, o_ref):
    r = r_ref[...]
    d2 = jnp.sum(r * r, axis=1, keepdims=True)
    d = jnp.sqrt(d2)
    width = np.float32(np.float32(CUTOFF) / (NRBF - 1))
    coeff = np.float32(-0.5 / (width * width))
    offsets = lax.broadcasted_iota(jnp.int32, (1, NRBF), 1).astype(jnp.float32) * width
    f = jnp.exp(coeff * (d - offsets) ** 2)
    t = _ssp(jnp.dot(f, w1_ref[...], precision=_DEF) + b1_ref[...])
    t = jnp.dot(t, w2_ref[...], precision=_DEF) + b2_ref[...]
    z = d2 * np.float32(1.0 / (CUTOFF * CUTOFF))
    p = jnp.full_like(z, _RCUT_COEF[-1])
    for c in _RCUT_COEF[-2::-1]:
        p = p * z + c
    rcut = 0.5 * (p + 1.0) * (z < 1.0).astype(jnp.float32)
    o_ref[...] = t * rcut


def _wij_call(r_ij, w1, b1, w2, b2):
    grid = (N_EDGES // BE_TC,)
    return pl.pallas_call(
        _wij_body,
        grid=grid,
        in_specs=[
            pl.BlockSpec((BE_TC, 3), lambda i: (i, 0)),
            pl.BlockSpec((NRBF, NF), lambda i: (0, 0)),
            pl.BlockSpec((1, NF), lambda i: (0, 0)),
            pl.BlockSpec((NF, NF), lambda i: (0, 0)),
            pl.BlockSpec((1, NF), lambda i: (0, 0)),
        ],
        out_specs=pl.BlockSpec((BE_TC, NF), lambda i: (i, 0)),
        out_shape=jax.ShapeDtypeStruct((N_EDGES, NF), jnp.float32),
        compiler_params=_MEGACORE,
    )(r_ij, w1, b1.reshape(1, NF), w2, b2.reshape(1, NF))


@functools.lru_cache(maxsize=None)
def _make_sc_fused():
    mesh = plsc.VectorSubcoreMesh(
        core_axis_name="c", subcore_axis_name="s",
        num_cores=SC_CORES, num_subcores=SC_SUBCORES,
    )
    return pl.kernel(
        _sc_fused_body,
        out_type=jax.ShapeDtypeStruct((SC_CORES, N_ATOMS, D), jnp.float32),
        mesh=mesh,
        scratch_types=[
            pltpu.VMEM_SHARED((N_ATOMS, D), jnp.float32),
            pltpu.VMEM((BE,), jnp.int32),
            pltpu.VMEM((BE,), jnp.int32),
            pltpu.VMEM((BE,), jnp.int32),
            pltpu.VMEM((BE,), jnp.int32),
            pltpu.VMEM((BE, D), jnp.float32),
            pltpu.VMEM((BE, D), jnp.float32),
            pltpu.VMEM((BE, D), jnp.float32),
            pltpu.VMEM((BE, D), jnp.float32),
            pltpu.SemaphoreType.DMA,
            pltpu.SemaphoreType.DMA,
            pltpu.SemaphoreType.DMA,
            pltpu.SemaphoreType.DMA,
            pltpu.SemaphoreType.DMA,
            pltpu.SemaphoreType.DMA,
        ],
    )


def _sc_call(h, wij, idx_i, idx_j, zeros):
    wij4 = wij.reshape(SC_CORES, SC_SUBCORES, NBLK, BE, NF)
    idxi4 = idx_i.reshape(SC_CORES, SC_SUBCORES, NBLK, BE)
    idxj4 = idx_j.reshape(SC_CORES, SC_SUBCORES, NBLK, BE)
    return _make_sc_fused()(h, wij4, idxi4, idxj4, zeros)


def _sc_fused_body(h_hbm, wij_hbm, idxi_hbm, idxj_hbm, zero_hbm, out_hbm,
                   agg_sh, idxi0, idxi1, idxj0, idxj1,
                   rows0, rows1, wijb0, wijb1,
                   sg0, sg1, sw0, sw1, si0, si1):
    cid = lax.axis_index("c")
    sid = lax.axis_index("s")
    idxi = (idxi0, idxi1)
    idxj = (idxj0, idxj1)
    rows = (rows0, rows1)
    wijb = (wijb0, wijb1)
    sg = (sg0, sg1)
    sw = (sw0, sw1)
    si = (si0, si1)

    pltpu.sync_copy(zero_hbm, agg_sh.at[pl.ds(sid * ROWS_PER_SUB, ROWS_PER_SUB)])

    @pl.when(sid == SC_SUBCORES - 1)
    def _zero_tail():
        pltpu.sync_copy(zero_hbm.at[pl.ds(0, ROWS_TAIL)],
                        agg_sh.at[pl.ds(SC_SUBCORES * ROWS_PER_SUB, ROWS_TAIL)])

    plsc.subcore_barrier()

    def start_idx(blk, p):
        pltpu.async_copy(idxi_hbm.at[cid, sid, blk], idxi[p], si[p])
        pltpu.async_copy(idxj_hbm.at[cid, sid, blk], idxj[p], si[p])

    def wait_idx(blk, p):
        pltpu.make_async_copy(idxi_hbm.at[cid, sid, blk], idxi[p], si[p]).wait()
        pltpu.make_async_copy(idxj_hbm.at[cid, sid, blk], idxj[p], si[p]).wait()

    def start_gw(blk, p):
        pltpu.async_copy(h_hbm.at[idxj[p]], rows[p], sg[p])
        pltpu.async_copy(wij_hbm.at[cid, sid, blk], wijb[p], sw[p])

    def wait_gw(blk, p):
        pltpu.make_async_copy(h_hbm.at[idxj[p]], rows[p], sg[p]).wait()
        pltpu.make_async_copy(wij_hbm.at[cid, sid, blk], wijb[p], sw[p]).wait()

    def process(blk, p, static_last=False):
        if not static_last:
            @pl.when(blk + 1 < NBLK)
            def _gw_next():
                wait_idx(blk + 1, 1 - p)
                start_gw(blk + 1, 1 - p)

        wait_gw(blk, p)

        @plsc.parallel_loop(0, BE, unroll=2)
        def _row(i):
            for j in range(D // 16):
                sl = (i, pl.ds(j * 16, 16))
                rows[p][sl] = rows[p][sl] * wijb[p][sl]

        pltpu.sync_copy(rows[p], agg_sh.at[idxi[p]], add=True)

        if not static_last:
            @pl.when(blk + 2 < NBLK)
            def _idx_next():
                start_idx(blk + 2, p)

    start_idx(0, 0)
    start_idx(1, 1)
    wait_idx(0, 0)
    start_gw(0, 0)

    @pl.loop(0, NBLK - 1, step=2)
    def _block(b):
        for p in range(2):
            process(b + p, p)

    process(NBLK - 1, 0, static_last=True)
    plsc.subcore_barrier()
    pltpu.sync_copy(
        agg_sh.at[pl.ds(sid * ROWS_PER_SUB, ROWS_PER_SUB)],
        out_hbm.at[cid, pl.ds(sid * ROWS_PER_SUB, ROWS_PER_SUB)],
    )

    @pl.when(sid == SC_SUBCORES - 1)
    def _write_tail():
        pltpu.sync_copy(
            agg_sh.at[pl.ds(SC_SUBCORES * ROWS_PER_SUB, ROWS_TAIL)],
            out_hbm.at[cid, pl.ds(SC_SUBCORES * ROWS_PER_SUB, ROWS_TAIL)],
        )


def _out_body(p_ref, x_ref, w1_ref, b1_ref, w2_ref, b2_ref, wn_ref, xo_ref, ho_ref):
    agg = p_ref[0] + p_ref[1]
    t = _ssp(jnp.dot(agg, w1_ref[...], precision=_HI) + b1_ref[...])
    v = jnp.dot(t, w2_ref[...], precision=_HI) + b2_ref[...]
    xn = x_ref[...] + v
    xo_ref[...] = xn
    ho_ref[...] = jnp.dot(xn, wn_ref[...], precision=_HI)


def _out_last_body(p_ref, x_ref, w1_ref, b1_ref, w2_ref, b2_ref, xo_ref):
    agg = p_ref[0] + p_ref[1]
    t = _ssp(jnp.dot(agg, w1_ref[...], precision=_HI) + b1_ref[...])
    v = jnp.dot(t, w2_ref[...], precision=_HI) + b2_ref[...]
    xo_ref[...] = x_ref[...] + v


def _out_call(partials, x, w1, b1, w2, b2, wn):
    grid = (N_ATOMS // BN,)
    common_in = [
        pl.BlockSpec((SC_CORES, BN, D), lambda i: (0, i, 0)),
        pl.BlockSpec((BN, D), lambda i: (i, 0)),
        pl.BlockSpec((NF, D), lambda i: (0, 0)),
        pl.BlockSpec((1, D), lambda i: (0, 0)),
        pl.BlockSpec((D, D), lambda i: (0, 0)),
        pl.BlockSpec((1, D), lambda i: (0, 0)),
    ]
    if wn is None:
        return pl.pallas_call(
            _out_last_body,
            grid=grid,
            in_specs=common_in,
            out_specs=pl.BlockSpec((BN, D), lambda i: (i, 0)),
            out_shape=jax.ShapeDtypeStruct((N_ATOMS, D), jnp.float32),
            compiler_params=_MEGACORE,
        )(partials, x, w1, b1.reshape(1, D), w2, b2.reshape(1, D))
    return pl.pallas_call(
        _out_body,
        grid=grid,
        in_specs=common_in + [pl.BlockSpec((D, NF), lambda i: (0, 0))],
        out_specs=[
            pl.BlockSpec((BN, D), lambda i: (i, 0)),
            pl.BlockSpec((BN, NF), lambda i: (i, 0)),
        ],
        out_shape=[
            jax.ShapeDtypeStruct((N_ATOMS, D), jnp.float32),
            jax.ShapeDtypeStruct((N_ATOMS, NF), jnp.float32),
        ],
        compiler_params=_MEGACORE,
    )(partials, x, w1, b1.reshape(1, D), w2, b2.reshape(1, D), wn)


def kernel(atomic_numbers, r_ij, idx_i, idx_j, emb,
           W_in2f, W_fn1, b_fn1, W_fn2, b_fn2,
           W_f2o1, b_f2o1, W_f2o2, b_f2o2):
    L = W_in2f.shape[0]
    z3 = atomic_numbers.astype(jnp.int32).reshape(N_ATOMS // BN, 1, BN)
    idx_i = idx_i.astype(jnp.int32)
    idx_j = idx_j.astype(jnp.int32)
    zeros = jnp.zeros((ROWS_PER_SUB, D), jnp.float32)

    x, h = _embed_call(z3, emb, W_in2f[0])
    for l in range(L):
        wij = _wij_call(r_ij, W_fn1[l], b_fn1[l], W_fn2[l], b_fn2[l])
        partials = _sc_call(h, wij, idx_i, idx_j, zeros)
        if l < L - 1:
            x, h = _out_call(partials, x, W_f2o1[l], b_f2o1[l],
                             W_f2o2[l], b_f2o2[l], W_in2f[l + 1])
        else:
            x = _out_call(partials, x, W_f2o1[l], b_f2o1[l],
                          W_f2o2[l], b_f2o2[l], None)
    return x


# speedup vs baseline: 1.1483x; 1.0034x over previous
import functools
import math

import jax
import jax.numpy as jnp
import numpy as np
from jax import lax
from jax.experimental import pallas as pl
from jax.experimental.pallas import tpu as pltpu
from jax.experimental.pallas import tpu_sc as plsc

N_ATOMS = 10000
N_EDGES = 320000
D = 128
NF = 128
NRBF = 20
MAXZ = 101
CUTOFF = 5.0

SC_CORES = 2
SC_SUBCORES = 16
EDGES_PER_CORE = N_EDGES // SC_CORES
EDGES_PER_SUB = EDGES_PER_CORE // SC_SUBCORES
BE = 80
NBLK = EDGES_PER_SUB // BE
ROWS_PER_SUB = 624
ROWS_TAIL = N_ATOMS - ROWS_PER_SUB * SC_SUBCORES

BN = 1000
BE_TC = 4000

_HI = jax.lax.Precision.HIGHEST
_DEF = jax.lax.Precision.DEFAULT
_MEGACORE = pltpu.CompilerParams(dimension_semantics=("parallel",))


def _ssp(x):
    return jnp.maximum(x, 0.0) + jnp.log(1.0 + jnp.exp(-jnp.abs(x))) - np.float32(np.log(2.0))


def _embed_body(z_ref, emb_ref, w_ref, x_ref, h_ref):
    z = z_ref[0, 0, :]
    oh = (z[:, None] == lax.broadcasted_iota(jnp.int32, (BN, MAXZ), 1)).astype(jnp.float32)
    x = jnp.dot(oh, emb_ref[...], precision=_HI)
    x_ref[...] = x
    h_ref[...] = jnp.dot(x, w_ref[...], precision=_HI)


def _embed_call(z3, emb, w0):
    grid = (N_ATOMS // BN,)
    return pl.pallas_call(
        _embed_body,
        grid=grid,
        in_specs=[
            pl.BlockSpec((1, 1, BN), lambda i: (i, 0, 0)),
            pl.BlockSpec((MAXZ, D), lambda i: (0, 0)),
            pl.BlockSpec((D, NF), lambda i: (0, 0)),
        ],
        out_specs=[
            pl.BlockSpec((BN, D), lambda i: (i, 0)),
            pl.BlockSpec((BN, NF), lambda i: (i, 0)),
        ],
        out_shape=[
            jax.ShapeDtypeStruct((N_ATOMS, D), jnp.float32),
            jax.ShapeDtypeStruct((N_ATOMS, NF), jnp.float32),
        ],
        compiler_params=_MEGACORE,
    )(z3, emb, w0)


_RCUT_COEF = tuple(np.float32(v) for v in (
    1.0, -4.93480110168457, 4.058694839477539, -1.3351584672927856,
    0.23502980172634125, -0.025358984246850014, 0.0015939107397571206,
))


def _wij_body(r_ref, w1_ref, b1_ref, w2_ref, b2_ref, o_ref):
    r = r_ref[...]
    d2 = jnp.sum(r * r, axis=1, keepdims=True)
    d = jnp.sqrt(d2)
    width = np.float32(np.float32(CUTOFF) / (NRBF - 1))
    coeff = np.float32(-0.5 / (width * width))
    offsets = lax.broadcasted_iota(jnp.int32, (1, NRBF), 1).astype(jnp.float32) * width
    f = jnp.exp(coeff * (d - offsets) ** 2)
    t = _ssp(jnp.dot(f, w1_ref[...], precision=_DEF) + b1_ref[...])
    t = jnp.dot(t, w2_ref[...], precision=_DEF) + b2_ref[...]
    z = d2 * np.float32(1.0 / (CUTOFF * CUTOFF))
    p = jnp.full_like(z, _RCUT_COEF[-1])
    for c in _RCUT_COEF[-2::-1]:
        p = p * z + c
    rcut = 0.5 * (p + 1.0) * (z < 1.0).astype(jnp.float32)
    o_ref[...] = t * rcut


def _wij_call(r_ij, w1, b1, w2, b2):
    grid = (N_EDGES // BE_TC,)
    return pl.pallas_call(
        _wij_body,
        grid=grid,
        in_specs=[
            pl.BlockSpec((BE_TC, 3), lambda i: (i, 0)),
            pl.BlockSpec((NRBF, NF), lambda i: (0, 0)),
            pl.BlockSpec((1, NF), lambda i: (0, 0)),
            pl.BlockSpec((NF, NF), lambda i: (0, 0)),
            pl.BlockSpec((1, NF), lambda i: (0, 0)),
        ],
        out_specs=pl.BlockSpec((BE_TC, NF), lambda i: (i, 0)),
        out_shape=jax.ShapeDtypeStruct((N_EDGES, NF), jnp.float32),
        compiler_params=_MEGACORE,
    )(r_ij, w1, b1.reshape(1, NF), w2, b2.reshape(1, NF))


@functools.lru_cache(maxsize=None)
def _make_sc_fused():
    mesh = plsc.VectorSubcoreMesh(
        core_axis_name="c", subcore_axis_name="s",
        num_cores=SC_CORES, num_subcores=SC_SUBCORES,
    )
    return pl.kernel(
        _sc_fused_body,
        out_type=jax.ShapeDtypeStruct((SC_CORES, N_ATOMS, D), jnp.float32),
        mesh=mesh,
        scratch_types=[
            pltpu.VMEM_SHARED((N_ATOMS, D), jnp.float32),
            pltpu.VMEM((BE,), jnp.int32),
            pltpu.VMEM((BE,), jnp.int32),
            pltpu.VMEM((BE,), jnp.int32),
            pltpu.VMEM((BE,), jnp.int32),
            pltpu.VMEM((BE, D), jnp.float32),
            pltpu.VMEM((BE, D), jnp.float32),
            pltpu.VMEM((BE, D), jnp.float32),
            pltpu.VMEM((BE, D), jnp.float32),
            pltpu.SemaphoreType.DMA,
            pltpu.SemaphoreType.DMA,
            pltpu.SemaphoreType.DMA,
            pltpu.SemaphoreType.DMA,
            pltpu.SemaphoreType.DMA,
            pltpu.SemaphoreType.DMA,
        ],
    )


def _sc_call(h, wij, idx_i, idx_j, zeros):
    wij4 = wij.reshape(SC_CORES, SC_SUBCORES, NBLK, BE, NF)
    idxi4 = idx_i.reshape(SC_CORES, SC_SUBCORES, NBLK, BE)
    idxj4 = idx_j.reshape(SC_CORES, SC_SUBCORES, NBLK, BE)
    return _make_sc_fused()(h, wij4, idxi4, idxj4, zeros)


def _sc_fused_body(h_hbm, wij_hbm, idxi_hbm, idxj_hbm, zero_hbm, out_hbm,
                   agg_sh, idxi0, idxi1, idxj0, idxj1,
                   rows0, rows1, wijb0, wijb1,
                   sg0, sg1, sw0, sw1, si0, si1):
    cid = lax.axis_index("c")
    sid = lax.axis_index("s")
    idxi = (idxi0, idxi1)
    idxj = (idxj0, idxj1)
    rows = (rows0, rows1)
    wijb = (wijb0, wijb1)
    sg = (sg0, sg1)
    sw = (sw0, sw1)
    si = (si0, si1)

    pltpu.sync_copy(zero_hbm, agg_sh.at[pl.ds(sid * ROWS_PER_SUB, ROWS_PER_SUB)])

    @pl.when(sid == SC_SUBCORES - 1)
    def _zero_tail():
        pltpu.sync_copy(zero_hbm.at[pl.ds(0, ROWS_TAIL)],
                        agg_sh.at[pl.ds(SC_SUBCORES * ROWS_PER_SUB, ROWS_TAIL)])

    plsc.subcore_barrier()

    def start_idx(blk, p):
        pltpu.async_copy(idxi_hbm.at[cid, sid, blk], idxi[p], si[p])
        pltpu.async_copy(idxj_hbm.at[cid, sid, blk], idxj[p], si[p])

    def wait_idx(blk, p):
        pltpu.make_async_copy(idxi_hbm.at[cid, sid, blk], idxi[p], si[p]).wait()
        pltpu.make_async_copy(idxj_hbm.at[cid, sid, blk], idxj[p], si[p]).wait()

    def start_gw(blk, p):
        pltpu.async_copy(h_hbm.at[idxj[p]], rows[p], sg[p])
        pltpu.async_copy(wij_hbm.at[cid, sid, blk], wijb[p], sw[p])

    def wait_gw(blk, p):
        pltpu.make_async_copy(h_hbm.at[idxj[p]], rows[p], sg[p]).wait()
        pltpu.make_async_copy(wij_hbm.at[cid, sid, blk], wijb[p], sw[p]).wait()

    def process(blk, p, static_last=False):
        if not static_last:
            @pl.when(blk + 1 < NBLK)
            def _gw_next():
                wait_idx(blk + 1, 1 - p)
                start_gw(blk + 1, 1 - p)

        wait_gw(blk, p)

        @plsc.parallel_loop(0, BE, unroll=4)
        def _row(i):
            for j in range(D // 16):
                sl = (i, pl.ds(j * 16, 16))
                rows[p][sl] = rows[p][sl] * wijb[p][sl]

        pltpu.sync_copy(rows[p], agg_sh.at[idxi[p]], add=True)

        if not static_last:
            @pl.when(blk + 2 < NBLK)
            def _idx_next():
                start_idx(blk + 2, p)

    start_idx(0, 0)
    start_idx(1, 1)
    wait_idx(0, 0)
    start_gw(0, 0)

    @pl.loop(0, NBLK - 1, step=2)
    def _block(b):
        for p in range(2):
            process(b + p, p)

    process(NBLK - 1, 0, static_last=True)
    plsc.subcore_barrier()
    pltpu.sync_copy(
        agg_sh.at[pl.ds(sid * ROWS_PER_SUB, ROWS_PER_SUB)],
        out_hbm.at[cid, pl.ds(sid * ROWS_PER_SUB, ROWS_PER_SUB)],
    )

    @pl.when(sid == SC_SUBCORES - 1)
    def _write_tail():
        pltpu.sync_copy(
            agg_sh.at[pl.ds(SC_SUBCORES * ROWS_PER_SUB, ROWS_TAIL)],
            out_hbm.at[cid, pl.ds(SC_SUBCORES * ROWS_PER_SUB, ROWS_TAIL)],
        )


def _out_body(p_ref, x_ref, w1_ref, b1_ref, w2_ref, b2_ref, wn_ref, xo_ref, ho_ref):
    agg = p_ref[0] + p_ref[1]
    t = _ssp(jnp.dot(agg, w1_ref[...], precision=_HI) + b1_ref[...])
    v = jnp.dot(t, w2_ref[...], precision=_HI) + b2_ref[...]
    xn = x_ref[...] + v
    xo_ref[...] = xn
    ho_ref[...] = jnp.dot(xn, wn_ref[...], precision=_HI)


def _out_last_body(p_ref, x_ref, w1_ref, b1_ref, w2_ref, b2_ref, xo_ref):
    agg = p_ref[0] + p_ref[1]
    t = _ssp(jnp.dot(agg, w1_ref[...], precision=_HI) + b1_ref[...])
    v = jnp.dot(t, w2_ref[...], precision=_HI) + b2_ref[...]
    xo_ref[...] = x_ref[...] + v


def _out_call(partials, x, w1, b1, w2, b2, wn):
    grid = (N_ATOMS // BN,)
    common_in = [
        pl.BlockSpec((SC_CORES, BN, D), lambda i: (0, i, 0)),
        pl.BlockSpec((BN, D), lambda i: (i, 0)),
        pl.BlockSpec((NF, D), lambda i: (0, 0)),
        pl.BlockSpec((1, D), lambda i: (0, 0)),
        pl.BlockSpec((D, D), lambda i: (0, 0)),
        pl.BlockSpec((1, D), lambda i: (0, 0)),
    ]
    if wn is None:
        return pl.pallas_call(
            _out_last_body,
            grid=grid,
            in_specs=common_in,
            out_specs=pl.BlockSpec((BN, D), lambda i: (i, 0)),
            out_shape=jax.ShapeDtypeStruct((N_ATOMS, D), jnp.float32),
            compiler_params=_MEGACORE,
        )(partials, x, w1, b1.reshape(1, D), w2, b2.reshape(1, D))
    return pl.pallas_call(
        _out_body,
        grid=grid,
        in_specs=common_in + [pl.BlockSpec((D, NF), lambda i: (0, 0))],
        out_specs=[
            pl.BlockSpec((BN, D), lambda i: (i, 0)),
            pl.BlockSpec((BN, NF), lambda i: (i, 0)),
        ],
        out_shape=[
            jax.ShapeDtypeStruct((N_ATOMS, D), jnp.float32),
            jax.ShapeDtypeStruct((N_ATOMS, NF), jnp.float32),
        ],
        compiler_params=_MEGACORE,
    )(partials, x, w1, b1.reshape(1, D), w2, b2.reshape(1, D), wn)


def kernel(atomic_numbers, r_ij, idx_i, idx_j, emb,
           W_in2f, W_fn1, b_fn1, W_fn2, b_fn2,
           W_f2o1, b_f2o1, W_f2o2, b_f2o2):
    L = W_in2f.shape[0]
    z3 = atomic_numbers.astype(jnp.int32).reshape(N_ATOMS // BN, 1, BN)
    idx_i = idx_i.astype(jnp.int32)
    idx_j = idx_j.astype(jnp.int32)
    zeros = jnp.zeros((ROWS_PER_SUB, D), jnp.float32)

    x, h = _embed_call(z3, emb, W_in2f[0])
    for l in range(L):
        wij = _wij_call(r_ij, W_fn1[l], b_fn1[l], W_fn2[l], b_fn2[l])
        partials = _sc_call(h, wij, idx_i, idx_j, zeros)
        if l < L - 1:
            x, h = _out_call(partials, x, W_f2o1[l], b_f2o1[l],
                             W_f2o2[l], b_f2o2[l], W_in2f[l + 1])
        else:
            x = _out_call(partials, x, W_f2o1[l], b_f2o1[l],
                          W_f2o2[l], b_f2o2[l], None)
    return x
